# Initial kernel scaffold; baseline (speedup 1.0000x reference)
import jax
import jax.numpy as jnp
from jax.experimental import pallas as pl


def kernel(x, edge_index, edge_attr, batch, params):
    raise NotImplementedError("write your pallas kernel here")


# speedup vs baseline: 2.5149x; 2.5149x over previous
import functools

import jax
import jax.numpy as jnp
from jax import lax
from jax.experimental import pallas as pl
from jax.experimental.pallas import tpu as pltpu
from jax.experimental.pallas import tpu_sc as plsc

N = 10000
E = 320000
DE = 16
H = 128
G = 64

NC = 2
NS = 16
NW = NC * NS
EPW = E // NW
CB = 80
NCHUNK = EPW // CB
CR = 200
NCOPY = N // CR

BR = 2000
BE = 4000


def _make_sc_layer(layer):
  mesh = plsc.VectorSubcoreMesh(core_axis_name="c", subcore_axis_name="s")

  @functools.partial(
      pl.kernel,
      mesh=mesh,
      out_type=jax.ShapeDtypeStruct((NC, N, H), jnp.float32),
      scratch_types=[
          pltpu.VMEM((CB,), jnp.int32),
          pltpu.VMEM((CB,), jnp.int32),
          pltpu.VMEM((CB, H), jnp.float32),
          pltpu.VMEM((CB, H), jnp.float32),
          pltpu.VMEM((CR, H), jnp.float32),
          pltpu.VMEM_SHARED((N, H), jnp.float32),
          pltpu.SemaphoreType.DMA,
      ],
  )
  def sc_layer(a_hbm, src_hbm, dst_hbm, eb_hbm, zeros_hbm, out_hbm,
               src_v, dst_v, rows_v, eb_v, zbuf_v, aggr_sh, sem):
    c = lax.axis_index("c")
    s = lax.axis_index("s")
    wid = s * NC + c

    pltpu.sync_copy(zeros_hbm, zbuf_v)
    for t in range((NCOPY + NS - 1) // NS):
      j = s + NS * t

      @pl.when(j < NCOPY)
      def _():
        pltpu.sync_copy(zbuf_v, aggr_sh.at[pl.ds(j * CR, CR)])

    plsc.subcore_barrier()

    base0 = wid * EPW

    def chunk(i, carry):
      base = base0 + i * CB
      pltpu.sync_copy(src_hbm.at[pl.ds(base, CB)], src_v)
      pltpu.sync_copy(dst_hbm.at[pl.ds(base, CB)], dst_v)
      pltpu.async_copy(a_hbm.at[src_v], rows_v, sem).wait()
      pltpu.sync_copy(eb_hbm.at[layer, pl.ds(base, CB)], eb_v)

      def edge(e, carry2):
        for k in range(H // 16):
          sl = pl.ds(k * 16, 16)
          v = rows_v[e, sl] + eb_v[e, sl]
          rows_v[e, sl] = jnp.maximum(v, 0.0)
        return carry2

      lax.fori_loop(0, CB, edge, 0)
      pltpu.sync_copy(rows_v, aggr_sh.at[dst_v], add=True)
      return carry

    lax.fori_loop(0, NCHUNK, chunk, 0)
    plsc.subcore_barrier()

    for t in range((NCOPY + NS - 1) // NS):
      j = s + NS * t

      @pl.when(j < NCOPY)
      def _():
        pltpu.sync_copy(aggr_sh.at[pl.ds(j * CR, CR)], zbuf_v)
        pltpu.sync_copy(zbuf_v, out_hbm.at[c, pl.ds(j * CR, CR)])

  return sc_layer


_SC_LAYERS = [_make_sc_layer(l) for l in range(3)]


def _eb_body(ea_ref, w_ref, c_ref, out_ref):
  out_ref[0] = (
      jnp.dot(ea_ref[...], w_ref[0], preferred_element_type=jnp.float32)
      + c_ref[0]
  )


def _edge_emb(ea, w_all, c_all):
  return pl.pallas_call(
      _eb_body,
      grid=(3, E // BE),
      in_specs=[
          pl.BlockSpec((BE, DE), lambda l, i: (i, 0)),
          pl.BlockSpec((1, DE, H), lambda l, i: (l, 0, 0)),
          pl.BlockSpec((1, 1, H), lambda l, i: (l, 0, 0)),
      ],
      out_specs=pl.BlockSpec((1, BE, H), lambda l, i: (l, i, 0)),
      out_shape=jax.ShapeDtypeStruct((3, E, H), jnp.float32),
  )(ea, w_all, c_all)


def _pre_body(x_ref, f_ref, out_ref):
  out_ref[...] = jnp.dot(x_ref[...], f_ref[...],
                         preferred_element_type=jnp.float32)


def _pre(x, f):
  return pl.pallas_call(
      _pre_body,
      grid=(N // BR,),
      in_specs=[
          pl.BlockSpec((BR, H), lambda i: (i, 0)),
          pl.BlockSpec((H, H), lambda i: (0, 0)),
      ],
      out_specs=pl.BlockSpec((BR, H), lambda i: (i, 0)),
      out_shape=jax.ShapeDtypeStruct((N, H), jnp.float32),
  )(x, f)


def _make_update(with_next):
  def body(h_ref, ag_ref, u_ref, b_ref, cu_ref, *rest):
    if with_next:
      fn_ref, h_out, a_out = rest
    else:
      (h_out,) = rest
    agg = ag_ref[0] + ag_ref[1]
    hb = jnp.maximum(
        jnp.dot(h_ref[...], u_ref[...], preferred_element_type=jnp.float32)
        + jnp.dot(agg, b_ref[...], preferred_element_type=jnp.float32)
        + cu_ref[...],
        0.0,
    )
    h_out[...] = hb
    if with_next:
      a_out[...] = jnp.dot(hb, fn_ref[...], preferred_element_type=jnp.float32)

  wspec = pl.BlockSpec((H, H), lambda i: (0, 0))
  in_specs = [
      pl.BlockSpec((BR, H), lambda i: (i, 0)),
      pl.BlockSpec((NC, BR, H), lambda i: (0, i, 0)),
      wspec,
      wspec,
      pl.BlockSpec((1, H), lambda i: (0, 0)),
  ]
  out_shape = [jax.ShapeDtypeStruct((N, H), jnp.float32)]
  out_specs = [pl.BlockSpec((BR, H), lambda i: (i, 0))]
  if with_next:
    in_specs.append(wspec)
    out_shape.append(jax.ShapeDtypeStruct((N, H), jnp.float32))
    out_specs.append(pl.BlockSpec((BR, H), lambda i: (i, 0)))

  def call(*args):
    return pl.pallas_call(
        body,
        grid=(N // BR,),
        in_specs=in_specs,
        out_specs=out_specs,
        out_shape=out_shape,
    )(*args)

  return call


_update_mid = _make_update(True)
_update_last = _make_update(False)


def _pool_body(h_ref, b_ref, w1_ref, b1_ref, w2_ref, b2_ref, out_ref, acc_ref):
  i = pl.program_id(0)

  @pl.when(i == 0)
  def _():
    acc_ref[...] = jnp.zeros_like(acc_ref)

  b = b_ref[0, 0, :]
  iota = lax.broadcasted_iota(jnp.int32, (BR, G), 1)
  onehot = (b[:, None] == iota).astype(jnp.float32)
  h_ext = jnp.concatenate(
      [h_ref[...], jnp.ones((BR, 1), jnp.float32)], axis=1)
  acc_ref[...] += lax.dot_general(
      onehot, h_ext, (((0,), (0,)), ((), ())),
      preferred_element_type=jnp.float32)

  @pl.when(i == (N // BR) - 1)
  def _():
    acc = acc_ref[...]
    pooled = acc[:, :H] / jnp.maximum(acc[:, H:H + 1], 1.0)
    h2 = jnp.maximum(
        jnp.dot(pooled, w1_ref[...], preferred_element_type=jnp.float32)
        + b1_ref[...],
        0.0,
    )
    out_ref[...] = (
        jnp.dot(h2, w2_ref[...], preferred_element_type=jnp.float32)
        + b2_ref[...]
    )


def _pool_head(h, batch_r, w1, b1, w2, b2):
  return pl.pallas_call(
      _pool_body,
      grid=(N // BR,),
      in_specs=[
          pl.BlockSpec((BR, H), lambda i: (i, 0)),
          pl.BlockSpec((1, 1, BR), lambda i: (i, 0, 0)),
          pl.BlockSpec((H, G), lambda i: (0, 0)),
          pl.BlockSpec((1, G), lambda i: (0, 0)),
          pl.BlockSpec((G, 1), lambda i: (0, 0)),
          pl.BlockSpec((1, 1), lambda i: (0, 0)),
      ],
      out_specs=pl.BlockSpec((G, 1), lambda i: (0, 0)),
      out_shape=jax.ShapeDtypeStruct((G, 1), jnp.float32),
      scratch_shapes=[pltpu.VMEM((G, H + 1), jnp.float32)],
  )(h, batch_r, w1, b1, w2, b2)


def _fold(p):
  wm = p["msg"]["W"]
  wm_top, wm_bot = wm[:H], wm[H:]
  f = p["sender"]["W"] @ wm_top
  wp = p["edge"]["W"] @ wm_bot
  cm = p["sender"]["b"] @ wm_top + p["edge"]["b"] @ wm_bot + p["msg"]["b"]
  wu = p["upd"]["W"]
  u = p["upd_x"]["W"] @ wu[:H]
  bmat = wu[H:]
  cu = p["upd_x"]["b"] @ wu[:H] + p["upd"]["b"]
  return f, wp, cm, u, bmat, cu


def kernel(x, edge_index, edge_attr, batch, params):
  src = edge_index[0]
  dst = edge_index[1]

  folds = [_fold(params[n]) for n in ("conv1", "conv2", "conv3")]
  wp_all = jnp.stack([f[1] for f in folds])
  c_all = jnp.stack([f[2] for f in folds])[:, None, :]
  zeros = jnp.zeros((CR, H), jnp.float32)

  eb_all = _edge_emb(edge_attr, wp_all, c_all)
  a = _pre(x, folds[0][0])

  h = x
  for l in range(3):
    _, _, _, u, bmat, cu = folds[l]
    aggr2 = _SC_LAYERS[l](a, src, dst, eb_all, zeros)
    if l < 2:
      h, a = _update_mid(h, aggr2, u, bmat, cu[None, :], folds[l + 1][0])
    else:
      (h,) = _update_last(h, aggr2, u, bmat, cu[None, :])

  out = _pool_head(
      h,
      batch.reshape(N // BR, 1, BR),
      params["fc1"]["W"],
      params["fc1"]["b"][None, :],
      params["fc2"]["W"],
      params["fc2"]["b"][None, :],
  )
  return out[:, 0]


# speedup vs baseline: 5.1271x; 2.0387x over previous
import functools

import jax
import jax.numpy as jnp
from jax import lax
from jax.experimental import pallas as pl
from jax.experimental.pallas import tpu as pltpu
from jax.experimental.pallas import tpu_sc as plsc

N = 10000
E = 320000
DE = 16
H = 128
G = 64

NC = 2
NS = 16
NW = NC * NS
EPW = E // NW
CB = 80
NCHUNK = EPW // CB
NCOPY = N // CB

BR = 2000
BE = 4000


def _make_sc_layer(layer):
  mesh = plsc.VectorSubcoreMesh(core_axis_name="c", subcore_axis_name="s")

  @functools.partial(
      pl.kernel,
      mesh=mesh,
      out_type=jax.ShapeDtypeStruct((NC, N, H), jnp.float32),
      scratch_types=[
          pltpu.VMEM((CB,), jnp.int32),
          pltpu.VMEM((CB,), jnp.int32),
          pltpu.VMEM((CB,), jnp.int32),
          pltpu.VMEM((CB,), jnp.int32),
          pltpu.VMEM((CB, H), jnp.float32),
          pltpu.VMEM((CB, H), jnp.float32),
          pltpu.VMEM((CB, H), jnp.float32),
          pltpu.VMEM((CB, H), jnp.float32),
          pltpu.VMEM_SHARED((N, H), jnp.float32),
      ] + [pltpu.SemaphoreType.DMA] * 10,
  )
  def sc_layer(a_hbm, src_hbm, dst_hbm, eb_hbm, zeros_hbm, out_hbm,
               src0, src1, dst0, dst1, rows0, rows1, ebv0, ebv1, aggr_sh,
               s_src0, s_src1, s_dst0, s_dst1, s_eb0, s_eb1,
               s_g0, s_g1, s_sc0, s_sc1):
    c = lax.axis_index("c")
    s = lax.axis_index("s")
    wid = s * NC + c
    src_v = (src0, src1)
    dst_v = (dst0, dst1)
    rows_v = (rows0, rows1)
    eb_v = (ebv0, ebv1)
    s_src = (s_src0, s_src1)
    s_dst = (s_dst0, s_dst1)
    s_eb = (s_eb0, s_eb1)
    s_g = (s_g0, s_g1)
    s_sc = (s_sc0, s_sc1)

    pltpu.sync_copy(zeros_hbm, rows0)
    for t in range((NCOPY + NS - 1) // NS):
      j = s + NS * t

      @pl.when(j < NCOPY)
      def _():
        pltpu.sync_copy(rows0, aggr_sh.at[pl.ds(j * CB, CB)])

    plsc.subcore_barrier()

    base0 = wid * EPW

    def load_src(i, b):
      pltpu.async_copy(src_hbm.at[pl.ds(base0 + i * CB, CB)],
                       src_v[b], s_src[b])

    def load_dst(i, b):
      pltpu.async_copy(dst_hbm.at[pl.ds(base0 + i * CB, CB)],
                       dst_v[b], s_dst[b])

    def load_eb(i, b):
      pltpu.async_copy(eb_hbm.at[layer, pl.ds(base0 + i * CB, CB)],
                       eb_v[b], s_eb[b])

    def compute(b, nb):
      def edge(e, carry):
        for k in range(H // 16):
          sl = pl.ds(k * 16, 16)
          v = rows_v[b][e, sl] + eb_v[b][e, sl]
          rows_v[b][e, sl] = jnp.maximum(v, 0.0)
        return carry

      lax.fori_loop(0, nb, edge, 0)

    def stage(i, b, first=False, a_wait=True, pre1=True, pre2=True,
              last=False):
      o = 1 - b
      if a_wait and not first:
        pltpu.make_async_copy(rows_v[o], aggr_sh.at[dst_v[o]], s_sc[o]).wait()
      if pre1:
        load_dst(i + 1, o)
        pltpu.make_async_copy(src_hbm.at[pl.ds(0, CB)], src_v[o],
                              s_src[o]).wait()
        pltpu.async_copy(a_hbm.at[src_v[o]], rows_v[o], s_g[o])
      pltpu.make_async_copy(a_hbm.at[pl.ds(0, CB)], rows_v[b], s_g[b]).wait()
      pltpu.make_async_copy(eb_hbm.at[layer, pl.ds(0, CB)], eb_v[b],
                            s_eb[b]).wait()
      if pre2:
        load_src(i + 2, b)
      compute(b, CB)
      if pre2:
        load_eb(i + 2, b)
      pltpu.make_async_copy(dst_hbm.at[pl.ds(0, CB)], dst_v[b],
                            s_dst[b]).wait()
      if last:
        pltpu.sync_copy(rows_v[b], aggr_sh.at[dst_v[b]], add=True)
      else:
        pltpu.async_copy(rows_v[b], aggr_sh.at[dst_v[b]], s_sc[b],
                         add=True)

    load_src(0, 0)
    load_src(1, 1)
    load_eb(0, 0)
    load_eb(1, 1)
    load_dst(0, 0)
    pltpu.make_async_copy(src_hbm.at[pl.ds(0, CB)], src_v[0],
                          s_src[0]).wait()
    pltpu.async_copy(a_hbm.at[src_v[0]], rows_v[0], s_g[0])
    stage(0, 0, first=True)
    stage(1, 1)
    stage(2, 0)

    def pair(g, carry):
      i1 = 2 * g + 3

      def full_pair():
        stage(i1, 1)
        stage(i1 + 1, 0)

      def last_pair():
        stage(i1, 1, pre2=False)
        stage(i1 + 1, 0, pre1=False, pre2=False, last=True)

      lax.cond(g < (NCHUNK - 3) // 2 - 1, full_pair, last_pair)
      return carry

    lax.fori_loop(0, (NCHUNK - 3) // 2, pair, 0)

    plsc.subcore_barrier()

    for t in range((NCOPY + NS - 1) // NS):
      j = s + NS * t

      @pl.when(j < NCOPY)
      def _():
        pltpu.sync_copy(aggr_sh.at[pl.ds(j * CB, CB)], rows0)
        pltpu.sync_copy(rows0, out_hbm.at[c, pl.ds(j * CB, CB)])

  return sc_layer


_SC_LAYERS = [_make_sc_layer(l) for l in range(3)]


def _eb_body(ea_ref, w_ref, c_ref, out_ref):
  out_ref[0] = (
      jnp.dot(ea_ref[...], w_ref[0], preferred_element_type=jnp.float32)
      + c_ref[0]
  )


def _edge_emb(ea, w_all, c_all):
  return pl.pallas_call(
      _eb_body,
      grid=(3, E // BE),
      in_specs=[
          pl.BlockSpec((BE, DE), lambda l, i: (i, 0)),
          pl.BlockSpec((1, DE, H), lambda l, i: (l, 0, 0)),
          pl.BlockSpec((1, 1, H), lambda l, i: (l, 0, 0)),
      ],
      out_specs=pl.BlockSpec((1, BE, H), lambda l, i: (l, i, 0)),
      out_shape=jax.ShapeDtypeStruct((3, E, H), jnp.float32),
  )(ea, w_all, c_all)


def _pre_body(x_ref, f_ref, out_ref):
  out_ref[...] = jnp.dot(x_ref[...], f_ref[...],
                         preferred_element_type=jnp.float32)


def _pre(x, f):
  return pl.pallas_call(
      _pre_body,
      grid=(N // BR,),
      in_specs=[
          pl.BlockSpec((BR, H), lambda i: (i, 0)),
          pl.BlockSpec((H, H), lambda i: (0, 0)),
      ],
      out_specs=pl.BlockSpec((BR, H), lambda i: (i, 0)),
      out_shape=jax.ShapeDtypeStruct((N, H), jnp.float32),
  )(x, f)


def _make_update(with_next):
  def body(h_ref, ag_ref, u_ref, b_ref, cu_ref, *rest):
    if with_next:
      fn_ref, h_out, a_out = rest
    else:
      (h_out,) = rest
    agg = ag_ref[0] + ag_ref[1]
    hb = jnp.maximum(
        jnp.dot(h_ref[...], u_ref[...], preferred_element_type=jnp.float32)
        + jnp.dot(agg, b_ref[...], preferred_element_type=jnp.float32)
        + cu_ref[...],
        0.0,
    )
    h_out[...] = hb
    if with_next:
      a_out[...] = jnp.dot(hb, fn_ref[...], preferred_element_type=jnp.float32)

  wspec = pl.BlockSpec((H, H), lambda i: (0, 0))
  in_specs = [
      pl.BlockSpec((BR, H), lambda i: (i, 0)),
      pl.BlockSpec((NC, BR, H), lambda i: (0, i, 0)),
      wspec,
      wspec,
      pl.BlockSpec((1, H), lambda i: (0, 0)),
  ]
  out_shape = [jax.ShapeDtypeStruct((N, H), jnp.float32)]
  out_specs = [pl.BlockSpec((BR, H), lambda i: (i, 0))]
  if with_next:
    in_specs.append(wspec)
    out_shape.append(jax.ShapeDtypeStruct((N, H), jnp.float32))
    out_specs.append(pl.BlockSpec((BR, H), lambda i: (i, 0)))

  def call(*args):
    return pl.pallas_call(
        body,
        grid=(N // BR,),
        in_specs=in_specs,
        out_specs=out_specs,
        out_shape=out_shape,
    )(*args)

  return call


_update_mid = _make_update(True)
_update_last = _make_update(False)


def _pool_body(h_ref, b_ref, w1_ref, b1_ref, w2_ref, b2_ref, out_ref, acc_ref):
  i = pl.program_id(0)

  @pl.when(i == 0)
  def _():
    acc_ref[...] = jnp.zeros_like(acc_ref)

  b = b_ref[0, 0, :]
  iota = lax.broadcasted_iota(jnp.int32, (BR, G), 1)
  onehot = (b[:, None] == iota).astype(jnp.float32)
  h_ext = jnp.concatenate(
      [h_ref[...], jnp.ones((BR, 1), jnp.float32)], axis=1)
  acc_ref[...] += lax.dot_general(
      onehot, h_ext, (((0,), (0,)), ((), ())),
      preferred_element_type=jnp.float32)

  @pl.when(i == (N // BR) - 1)
  def _():
    acc = acc_ref[...]
    pooled = acc[:, :H] / jnp.maximum(acc[:, H:H + 1], 1.0)
    h2 = jnp.maximum(
        jnp.dot(pooled, w1_ref[...], preferred_element_type=jnp.float32)
        + b1_ref[...],
        0.0,
    )
    out_ref[...] = (
        jnp.dot(h2, w2_ref[...], preferred_element_type=jnp.float32)
        + b2_ref[...]
    )


def _pool_head(h, batch_r, w1, b1, w2, b2):
  return pl.pallas_call(
      _pool_body,
      grid=(N // BR,),
      in_specs=[
          pl.BlockSpec((BR, H), lambda i: (i, 0)),
          pl.BlockSpec((1, 1, BR), lambda i: (i, 0, 0)),
          pl.BlockSpec((H, G), lambda i: (0, 0)),
          pl.BlockSpec((1, G), lambda i: (0, 0)),
          pl.BlockSpec((G, 1), lambda i: (0, 0)),
          pl.BlockSpec((1, 1), lambda i: (0, 0)),
      ],
      out_specs=pl.BlockSpec((G, 1), lambda i: (0, 0)),
      out_shape=jax.ShapeDtypeStruct((G, 1), jnp.float32),
      scratch_shapes=[pltpu.VMEM((G, H + 1), jnp.float32)],
  )(h, batch_r, w1, b1, w2, b2)


def _fold(p):
  wm = p["msg"]["W"]
  wm_top, wm_bot = wm[:H], wm[H:]
  f = p["sender"]["W"] @ wm_top
  wp = p["edge"]["W"] @ wm_bot
  cm = p["sender"]["b"] @ wm_top + p["edge"]["b"] @ wm_bot + p["msg"]["b"]
  wu = p["upd"]["W"]
  u = p["upd_x"]["W"] @ wu[:H]
  bmat = wu[H:]
  cu = p["upd_x"]["b"] @ wu[:H] + p["upd"]["b"]
  return f, wp, cm, u, bmat, cu


def kernel(x, edge_index, edge_attr, batch, params):
  src = edge_index[0]
  dst = edge_index[1]
  folds = [_fold(params[n]) for n in ("conv1", "conv2", "conv3")]
  wp_all = jnp.stack([f[1] for f in folds])
  c_all = jnp.stack([f[2] for f in folds])[:, None, :]
  zeros = jnp.zeros((CB, H), jnp.float32)

  eb_all = _edge_emb(edge_attr, wp_all, c_all)
  a = _pre(x, folds[0][0])

  h = x
  for l in range(3):
    _, _, _, u, bmat, cu = folds[l]
    aggr2 = _SC_LAYERS[l](a, src, dst, eb_all, zeros)
    if l < 2:
      h, a = _update_mid(h, aggr2, u, bmat, cu[None, :], folds[l + 1][0])
    else:
      (h,) = _update_last(h, aggr2, u, bmat, cu[None, :])

  out = _pool_head(
      h,
      batch.reshape(N // BR, 1, BR),
      params["fc1"]["W"],
      params["fc1"]["b"][None, :],
      params["fc2"]["W"],
      params["fc2"]["b"][None, :],
  )
  return out[:, 0]


# speedup vs baseline: 5.5945x; 1.0912x over previous
import functools

import numpy as np

import jax
import jax.numpy as jnp
from jax import lax
from jax.experimental import pallas as pl
from jax.experimental.pallas import tpu as pltpu
from jax.experimental.pallas import tpu_sc as plsc

N = 10000
E = 320000
DE = 16
H = 128
G = 64

NC = 2
NS = 16
NW = NC * NS
EPW = E // NW
CB = 80
NCHUNK = EPW // CB
NCOPY = N // CB

BR = 2000
BE = 4000


def _make_sc_layer():
  mesh = plsc.VectorSubcoreMesh(core_axis_name="c", subcore_axis_name="s")

  @functools.partial(
      pl.kernel,
      mesh=mesh,
      out_type=jax.ShapeDtypeStruct((NC, N, H), jnp.float32),
      scratch_types=[
          pltpu.VMEM((CB,), jnp.int32),
          pltpu.VMEM((CB,), jnp.int32),
          pltpu.VMEM((CB,), jnp.int32),
          pltpu.VMEM((CB,), jnp.int32),
          pltpu.VMEM((CB, H), jnp.float32),
          pltpu.VMEM((CB, H), jnp.float32),
          pltpu.VMEM((CB, H), jnp.float32),
          pltpu.VMEM((CB, H), jnp.float32),
          pltpu.VMEM_SHARED((N, H), jnp.float32),
      ] + [pltpu.SemaphoreType.DMA] * 10,
  )
  def sc_layer(a_hbm, src_hbm, dst_hbm, eb_hbm, zeros_hbm, out_hbm,
               src0, src1, dst0, dst1, rows0, rows1, ebv0, ebv1, aggr_sh,
               s_src0, s_src1, s_dst0, s_dst1, s_eb0, s_eb1,
               s_g0, s_g1, s_sc0, s_sc1):
    c = lax.axis_index("c")
    s = lax.axis_index("s")
    wid = s * NC + c
    src_v = (src0, src1)
    dst_v = (dst0, dst1)
    rows_v = (rows0, rows1)
    eb_v = (ebv0, ebv1)
    s_src = (s_src0, s_src1)
    s_dst = (s_dst0, s_dst1)
    s_eb = (s_eb0, s_eb1)
    s_g = (s_g0, s_g1)
    s_sc = (s_sc0, s_sc1)

    pltpu.sync_copy(zeros_hbm, rows0)
    for t in range((NCOPY + NS - 1) // NS):
      j = s + NS * t

      @pl.when(j < NCOPY)
      def _():
        pltpu.sync_copy(rows0, aggr_sh.at[pl.ds(j * CB, CB)])

    plsc.subcore_barrier()

    base0 = wid * EPW

    def load_src(i, b):
      pltpu.async_copy(src_hbm.at[pl.ds(base0 + i * CB, CB)],
                       src_v[b], s_src[b])

    def load_dst(i, b):
      pltpu.async_copy(dst_hbm.at[pl.ds(base0 + i * CB, CB)],
                       dst_v[b], s_dst[b])

    def load_eb(i, b):
      pltpu.async_copy(eb_hbm.at[pl.ds(base0 + i * CB, CB)],
                       eb_v[b], s_eb[b])

    def compute(b, nb):
      def edge(e, carry):
        for k in range(H // 16):
          sl = pl.ds(k * 16, 16)
          v = rows_v[b][e, sl] + eb_v[b][e, sl]
          rows_v[b][e, sl] = jnp.maximum(v, 0.0)
        return carry

      lax.fori_loop(0, nb, edge, 0)

    def stage(i, b, first=False, a_wait=True, pre1=True, pre2=True,
              last=False):
      o = 1 - b
      if a_wait and not first:
        pltpu.make_async_copy(rows_v[o], aggr_sh.at[dst_v[o]], s_sc[o]).wait()
      if pre1:
        load_dst(i + 1, o)
        pltpu.make_async_copy(src_hbm.at[pl.ds(0, CB)], src_v[o],
                              s_src[o]).wait()
        pltpu.async_copy(a_hbm.at[src_v[o]], rows_v[o], s_g[o])
      pltpu.make_async_copy(a_hbm.at[pl.ds(0, CB)], rows_v[b], s_g[b]).wait()
      pltpu.make_async_copy(eb_hbm.at[pl.ds(0, CB)], eb_v[b],
                            s_eb[b]).wait()
      if pre2:
        load_src(i + 2, b)
      compute(b, CB)
      if pre2:
        load_eb(i + 2, b)
      pltpu.make_async_copy(dst_hbm.at[pl.ds(0, CB)], dst_v[b],
                            s_dst[b]).wait()
      if last:
        pltpu.sync_copy(rows_v[b], aggr_sh.at[dst_v[b]], add=True)
      else:
        pltpu.async_copy(rows_v[b], aggr_sh.at[dst_v[b]], s_sc[b],
                         add=True)

    load_src(0, 0)
    load_src(1, 1)
    load_eb(0, 0)
    load_eb(1, 1)
    load_dst(0, 0)
    pltpu.make_async_copy(src_hbm.at[pl.ds(0, CB)], src_v[0],
                          s_src[0]).wait()
    pltpu.async_copy(a_hbm.at[src_v[0]], rows_v[0], s_g[0])
    stage(0, 0, first=True)
    stage(1, 1)
    stage(2, 0)

    def pair(g, carry):
      i1 = 2 * g + 3

      def full_pair():
        stage(i1, 1)
        stage(i1 + 1, 0)

      def last_pair():
        stage(i1, 1, pre2=False)
        stage(i1 + 1, 0, pre1=False, pre2=False, last=True)

      lax.cond(g < (NCHUNK - 3) // 2 - 1, full_pair, last_pair)
      return carry

    lax.fori_loop(0, (NCHUNK - 3) // 2, pair, 0)

    plsc.subcore_barrier()

    for t in range((NCOPY + NS - 1) // NS):
      j = s + NS * t

      @pl.when(j < NCOPY)
      def _():
        pltpu.sync_copy(aggr_sh.at[pl.ds(j * CB, CB)], rows0)
        pltpu.sync_copy(rows0, out_hbm.at[c, pl.ds(j * CB, CB)])

  return sc_layer


@functools.lru_cache(maxsize=None)
def _get_sc_layer():
  return _make_sc_layer()


def _eb_body(ea_ref, w_ref, c_ref, out_ref):
  out_ref[...] = (
      jnp.dot(ea_ref[...], w_ref[...], preferred_element_type=jnp.float32)
      + c_ref[...]
  )


def _edge_emb(ea, w, c):
  return pl.pallas_call(
      _eb_body,
      grid=(E // BE,),
      in_specs=[
          pl.BlockSpec((BE, DE), lambda i: (i, 0)),
          pl.BlockSpec((DE, H), lambda i: (0, 0)),
          pl.BlockSpec((1, H), lambda i: (0, 0)),
      ],
      out_specs=pl.BlockSpec((BE, H), lambda i: (i, 0)),
      out_shape=jax.ShapeDtypeStruct((E, H), jnp.float32),
  )(ea, w, c)


def _pre_body(x_ref, f_ref, out_ref):
  out_ref[...] = jnp.dot(x_ref[...], f_ref[...],
                         preferred_element_type=jnp.float32)


def _pre(x, f):
  return pl.pallas_call(
      _pre_body,
      grid=(N // BR,),
      in_specs=[
          pl.BlockSpec((BR, H), lambda i: (i, 0)),
          pl.BlockSpec((H, H), lambda i: (0, 0)),
      ],
      out_specs=pl.BlockSpec((BR, H), lambda i: (i, 0)),
      out_shape=jax.ShapeDtypeStruct((N, H), jnp.float32),
  )(x, f)


def _make_update(with_next):
  def body(h_ref, ag_ref, u_ref, b_ref, cu_ref, *rest):
    if with_next:
      fn_ref, h_out, a_out = rest
    else:
      (h_out,) = rest
    agg = ag_ref[0] + ag_ref[1]
    hb = jnp.maximum(
        jnp.dot(h_ref[...], u_ref[...], preferred_element_type=jnp.float32)
        + jnp.dot(agg, b_ref[...], preferred_element_type=jnp.float32)
        + cu_ref[...],
        0.0,
    )
    h_out[...] = hb
    if with_next:
      a_out[...] = jnp.dot(hb, fn_ref[...], preferred_element_type=jnp.float32)

  wspec = pl.BlockSpec((H, H), lambda i: (0, 0))
  in_specs = [
      pl.BlockSpec((BR, H), lambda i: (i, 0)),
      pl.BlockSpec((NC, BR, H), lambda i: (0, i, 0)),
      wspec,
      wspec,
      pl.BlockSpec((1, H), lambda i: (0, 0)),
  ]
  out_shape = [jax.ShapeDtypeStruct((N, H), jnp.float32)]
  out_specs = [pl.BlockSpec((BR, H), lambda i: (i, 0))]
  if with_next:
    in_specs.append(wspec)
    out_shape.append(jax.ShapeDtypeStruct((N, H), jnp.float32))
    out_specs.append(pl.BlockSpec((BR, H), lambda i: (i, 0)))

  def call(*args):
    return pl.pallas_call(
        body,
        grid=(N // BR,),
        in_specs=in_specs,
        out_specs=out_specs,
        out_shape=out_shape,
    )(*args)

  return call


_update_mid = _make_update(True)
_update_last = _make_update(False)


def _pool_body(h_ref, b_ref, w1_ref, b1_ref, w2_ref, b2_ref, out_ref, acc_ref):
  i = pl.program_id(0)

  @pl.when(i == 0)
  def _():
    acc_ref[...] = jnp.zeros_like(acc_ref)

  b = b_ref[0, 0, :]
  iota = lax.broadcasted_iota(jnp.int32, (BR, G), 1)
  onehot = (b[:, None] == iota).astype(jnp.float32)
  h_ext = jnp.concatenate(
      [h_ref[...], jnp.ones((BR, 1), jnp.float32)], axis=1)
  acc_ref[...] += lax.dot_general(
      onehot, h_ext, (((0,), (0,)), ((), ())),
      preferred_element_type=jnp.float32)

  @pl.when(i == (N // BR) - 1)
  def _():
    acc = acc_ref[...]
    pooled = acc[:, :H] / jnp.maximum(acc[:, H:H + 1], 1.0)
    h2 = jnp.maximum(
        jnp.dot(pooled, w1_ref[...], preferred_element_type=jnp.float32)
        + b1_ref[...],
        0.0,
    )
    out_ref[...] = (
        jnp.dot(h2, w2_ref[...], preferred_element_type=jnp.float32)
        + b2_ref[...]
    )


def _pool_head(h, batch_r, w1, b1, w2, b2):
  return pl.pallas_call(
      _pool_body,
      grid=(N // BR,),
      in_specs=[
          pl.BlockSpec((BR, H), lambda i: (i, 0)),
          pl.BlockSpec((1, 1, BR), lambda i: (i, 0, 0)),
          pl.BlockSpec((H, G), lambda i: (0, 0)),
          pl.BlockSpec((1, G), lambda i: (0, 0)),
          pl.BlockSpec((G, 1), lambda i: (0, 0)),
          pl.BlockSpec((1, 1), lambda i: (0, 0)),
      ],
      out_specs=pl.BlockSpec((G, 1), lambda i: (0, 0)),
      out_shape=jax.ShapeDtypeStruct((G, 1), jnp.float32),
      scratch_shapes=[pltpu.VMEM((G, H + 1), jnp.float32)],
  )(h, batch_r, w1, b1, w2, b2)


def _fold(p):
  wm = p["msg"]["W"]
  wm_top, wm_bot = wm[:H], wm[H:]
  f = p["sender"]["W"] @ wm_top
  wp = p["edge"]["W"] @ wm_bot
  cm = p["sender"]["b"] @ wm_top + p["edge"]["b"] @ wm_bot + p["msg"]["b"]
  wu = p["upd"]["W"]
  u = p["upd_x"]["W"] @ wu[:H]
  bmat = wu[H:]
  cu = p["upd_x"]["b"] @ wu[:H] + p["upd"]["b"]
  return f, wp, cm, u, bmat, cu


def kernel(x, edge_index, edge_attr, batch, params):
  src = edge_index[0]
  dst = edge_index[1]
  folds = [_fold(params[n]) for n in ("conv1", "conv2", "conv3")]
  zeros = jnp.zeros((CB, H), jnp.float32)

  ebs = [_edge_emb(edge_attr, f[1], f[2][None, :]) for f in folds]
  a = _pre(x, folds[0][0])

  h = x
  for l in range(3):
    _, _, _, u, bmat, cu = folds[l]
    aggr2 = _get_sc_layer()(a, src, dst, ebs[l], zeros)
    if l < 2:
      h, a = _update_mid(h, aggr2, u, bmat, cu[None, :], folds[l + 1][0])
    else:
      (h,) = _update_last(h, aggr2, u, bmat, cu[None, :])

  out = _pool_head(
      h,
      batch.reshape(N // BR, 1, BR),
      params["fc1"]["W"],
      params["fc1"]["b"][None, :],
      params["fc2"]["W"],
      params["fc2"]["b"][None, :],
  )
  return out[:, 0]
